# baseline (speedup 1.0000x reference)
import functools

import jax
import jax.numpy as jnp
import numpy as np
from jax import lax
from jax.experimental import pallas as pl
from jax.experimental.pallas import tpu as pltpu
from jax.experimental.pallas import tpu_sc as plsc

_N = 10000
_NP = 10240
_E = 320000
_F = 128
_W = 128
_HQ = 32
_NC = 2
_NS = 16
_NW = _NC * _NS
_CH = 128
_CHUNKS = 80
_EPW = _CHUNKS * _CH
_NE = _NW * _EPW
_RPS = _NP // _NS
_BLK = 1024
_B = 1024
_BPW = _B // _NW
_ECH = 1280

_mesh = plsc.VectorSubcoreMesh(core_axis_name="c", subcore_axis_name="s")


@functools.partial(
    pl.kernel,
    out_type=jax.ShapeDtypeStruct((_NW, _NP), jnp.float32),
    mesh=_mesh,
    scratch_types=[
        pltpu.VMEM((_NP,), jnp.float32),
        pltpu.VMEM((_ECH,), jnp.int32),
    ],
    compiler_params=pltpu.CompilerParams(needs_layout_passes=False),
)
def _deg_kernel(dst_hbm, out_hbm, hist, dst_v):
    c = lax.axis_index("c")
    s = lax.axis_index("s")
    wid = c * _NS + s
    zv = jnp.zeros((16,), jnp.float32)

    def zloop(i, carry):
        hist[pl.ds(i * 16, 16)] = zv
        return carry

    lax.fori_loop(0, _NP // 16, zloop, 0)

    ones = jnp.ones((16,), jnp.float32)
    base = wid * _EPW

    def outer(t, carry):
        pltpu.sync_copy(dst_hbm.at[pl.ds(base + t * _ECH, _ECH)], dst_v)

        def inner(j, carry2):
            idx = dst_v[pl.ds(j * 16, 16)]
            plsc.addupdate_scatter(hist, [idx], ones)
            return carry2

        lax.fori_loop(0, _ECH // 16, inner, 0)
        return carry

    lax.fori_loop(0, _EPW // _ECH, outer, 0)
    pltpu.sync_copy(hist, out_hbm.at[wid])


def _make_scatter(width, c0_chunks, c1_chunks):
    w32 = width // 2
    cmax = max(c0_chunks, c1_chunks)

    @functools.partial(
        pl.kernel,
        out_type=jax.ShapeDtypeStruct((_NC, _NP, width), jnp.float32),
        mesh=_mesh,
        scratch_types=[
            pltpu.VMEM_SHARED((_NP, width), jnp.float32),
            pltpu.VMEM((cmax, _CH), jnp.int32),
            pltpu.VMEM((_CH,), jnp.int32),
            pltpu.VMEM((_CH,), jnp.int32),
            pltpu.VMEM((_CH, w32), jnp.int32),
            pltpu.VMEM((_CH, w32), jnp.int32),
            pltpu.VMEM((_CH, width), jnp.float32),
            pltpu.SemaphoreType.DMA,
            pltpu.SemaphoreType.DMA,
            pltpu.SemaphoreType.DMA,
            pltpu.SemaphoreType.DMA,
        ],
        compiler_params=pltpu.CompilerParams(
            use_tc_tiling_on_sc=False, needs_layout_passes=False),
    )
    def _scat(g_hbm, src_hbm, dst_hbm, out_hbm, acc, srcs, dstb0, dstb1,
              rows0, rows1, rowsf, gsem0, gsem1, dsem0, dsem1):
        c = lax.axis_index("c")
        s = lax.axis_index("s")
        wid = c * _NS + s
        nchunks = jnp.where(c == 0, c0_chunks, c1_chunks)

        pltpu.sync_copy(src_hbm.at[wid], srcs)

        zv = jnp.zeros((16,), jnp.float32)
        for i in range(16):
            for j in range(width // 16):
                rowsf[i, 16 * j:16 * (j + 1)] = zv

        def zloop(k, carry):
            pltpu.sync_copy(rowsf.at[pl.ds(0, 16)],
                            acc.at[pl.ds(s * _RPS + k * 16, 16)])
            return carry

        lax.fori_loop(0, _RPS // 16, zloop, 0)
        plsc.subcore_barrier()

        pltpu.async_copy(dst_hbm.at[wid, 0], dstb0, dsem0)
        pltpu.async_copy(g_hbm.at[srcs.at[0]], rows0, gsem0)

        def body(i, carry):
            for b, rows, gsem, dstb, dsem, orows, ogsem, odstb, odsem in (
                (0, rows0, gsem0, dstb0, dsem0, rows1, gsem1, dstb1, dsem1),
                (1, rows1, gsem1, dstb1, dsem1, rows0, gsem0, dstb0, dsem0),
            ):
                t = 2 * i + b
                pltpu.make_async_copy(dst_hbm.at[wid, t], dstb, dsem).wait()
                pltpu.make_async_copy(g_hbm.at[srcs.at[t]], rows, gsem).wait()
                nt = t + 1

                @pl.when(nt < nchunks)
                def _():
                    pltpu.async_copy(dst_hbm.at[wid, nt], odstb, odsem)
                    pltpu.async_copy(g_hbm.at[srcs.at[nt]], orows, ogsem)

                def conv(r, carry2):
                    for j in range(w32 // 16):
                        v = rows[r, 16 * j:16 * (j + 1)]
                        rowsf[r, 32 * j:32 * j + 16] = plsc.bitcast(
                            v << 16, jnp.float32)
                        rowsf[r, 32 * j + 16:32 * j + 32] = plsc.bitcast(
                            v & jnp.int32(-65536), jnp.float32)
                    return carry2

                lax.fori_loop(0, _CH, conv, 0)
                pltpu.sync_copy(rowsf, acc.at[dstb], add=True)
            return carry

        lax.fori_loop(0, nchunks // 2, body, 0)
        plsc.subcore_barrier()
        pltpu.sync_copy(acc.at[pl.ds(s * _RPS, _RPS)],
                        out_hbm.at[c, pl.ds(s * _RPS, _RPS)])

    return _scat


_C0W, _C1W = 80, 80
_C0N, _C1N = 80, 80

_scatter_w = _make_scatter(_W, _C0W, _C1W)
_scatter_n = _make_scatter(_HQ, _C0N, _C1N)


def _mkperm(w):
    p = np.empty((w,), np.int32)
    for k in range(w // 32):
        for t in range(16):
            p[32 * k + 2 * t] = 32 * k + t
            p[32 * k + 2 * t + 1] = 32 * k + 16 + t
    return p


_PERMW = _mkperm(_W)
_PERMN = _mkperm(_HQ)


def _to_i32(gbf):
    n, w = gbf.shape
    return jax.lax.bitcast_convert_type(
        gbf.reshape(n, w // 2, 2), jnp.int32)


def _pack_edges(v, c0, c1):
    cmax = max(c0, c1)
    t0 = _NS * c0 * _CH
    a = v[:t0].reshape(_NS, c0, _CH)
    b = v[t0:].reshape(_NS, c1, _CH)
    a = jnp.pad(a, ((0, 0), (0, cmax - c0), (0, 0)), constant_values=_N)
    b = jnp.pad(b, ((0, 0), (0, cmax - c1), (0, 0)), constant_values=_N)
    return jnp.concatenate([a, b], axis=0)


@functools.partial(
    pl.kernel,
    out_type=jax.ShapeDtypeStruct((_B, _HQ), jnp.float32),
    mesh=_mesh,
    scratch_types=[
        pltpu.VMEM((_BPW,), jnp.int32),
        pltpu.VMEM((_BPW, _HQ), jnp.float32),
        pltpu.SemaphoreType.DMA,
    ],
    compiler_params=pltpu.CompilerParams(use_tc_tiling_on_sc=False),
)
def _gather_rows(h_hbm, idx_hbm, out_hbm, idx_v, rows_v, sem):
    c = lax.axis_index("c")
    s = lax.axis_index("s")
    wid = c * _NS + s
    pltpu.sync_copy(idx_hbm.at[pl.ds(wid * _BPW, _BPW)], idx_v)
    pltpu.async_copy(h_hbm.at[idx_v], rows_v, sem).wait()
    pltpu.sync_copy(rows_v, out_hbm.at[pl.ds(wid * _BPW, _BPW)])


def _scale_in_body(degp_ref, x_ref, w_ref, g_ref, dinv_ref):
    deg = jnp.sum(degp_ref[...], axis=0)[:, None] + 1.0
    dinv = lax.rsqrt(deg)
    i = pl.program_id(0)
    rows = i * _BLK + lax.broadcasted_iota(jnp.int32, (_BLK, 1), 0)
    h = jnp.dot(x_ref[...], w_ref[...], preferred_element_type=jnp.float32)
    g_ref[...] = jnp.where(rows < _N, h * dinv, 0.0).astype(jnp.bfloat16)
    dinv_ref[...] = dinv


_scale_in = pl.pallas_call(
    _scale_in_body,
    grid=(_NP // _BLK,),
    in_specs=[
        pl.BlockSpec((_NW, _BLK), lambda i: (0, i)),
        pl.BlockSpec((_BLK, _F), lambda i: (i, 0)),
        pl.BlockSpec((_F, _W), lambda i: (0, 0)),
    ],
    out_specs=[
        pl.BlockSpec((_BLK, _W), lambda i: (i, 0)),
        pl.BlockSpec((_BLK, 1), lambda i: (i, 0)),
    ],
    out_shape=[
        jax.ShapeDtypeStruct((_NP, _W), jnp.bfloat16),
        jax.ShapeDtypeStruct((_NP, 1), jnp.float32),
    ],
)


def _deinterleave(g):
    b, w = g.shape
    return g.reshape(b, w // 32, 16, 2).swapaxes(-1, -2).reshape(b, w)


def _make_combine_matmul(din, dout):
    def body(p_ref, g_ref, dinv_ref, b_ref, w_ref, out_ref):
        i = pl.program_id(0)
        p = p_ref[...]
        dinv = dinv_ref[...]
        g = _deinterleave(g_ref[...].astype(jnp.float32))
        u = dinv * (p[0] + p[1] + g) + b_ref[...]
        rows = i * _BLK + lax.broadcasted_iota(jnp.int32, (_BLK, 1), 0)
        v = jnp.where(rows < _N, jnp.maximum(u, 0.0), 0.0)
        out = jnp.dot(v, w_ref[...], preferred_element_type=jnp.float32)
        out_ref[...] = (out * dinv).astype(jnp.bfloat16)

    return pl.pallas_call(
        body,
        grid=(_NP // _BLK,),
        in_specs=[
            pl.BlockSpec((2, _BLK, din), lambda i: (0, i, 0)),
            pl.BlockSpec((_BLK, din), lambda i: (i, 0)),
            pl.BlockSpec((_BLK, 1), lambda i: (i, 0)),
            pl.BlockSpec((1, din), lambda i: (0, 0)),
            pl.BlockSpec((din, dout), lambda i: (0, 0)),
        ],
        out_specs=pl.BlockSpec((_BLK, dout), lambda i: (i, 0)),
        out_shape=jax.ShapeDtypeStruct((_NP, dout), jnp.bfloat16),
    )


_combine12 = _make_combine_matmul(_W, _HQ)
_combine23 = _make_combine_matmul(_HQ, _HQ)


def _combine_out_body(p_ref, g_ref, dinv_ref, b_ref, out_ref):
    p = p_ref[...]
    g = _deinterleave(g_ref[...].astype(jnp.float32))
    out_ref[...] = dinv_ref[...] * (p[0] + p[1] + g) + b_ref[...]


_combine_out = pl.pallas_call(
    _combine_out_body,
    grid=(_NP // _BLK,),
    in_specs=[
        pl.BlockSpec((2, _BLK, _HQ), lambda i: (0, i, 0)),
        pl.BlockSpec((_BLK, _HQ), lambda i: (i, 0)),
        pl.BlockSpec((_BLK, 1), lambda i: (i, 0)),
        pl.BlockSpec((1, _HQ), lambda i: (0, 0)),
    ],
    out_specs=pl.BlockSpec((_BLK, _HQ), lambda i: (i, 0)),
    out_shape=jax.ShapeDtypeStruct((_NP, _HQ), jnp.float32),
)


def _head_body(hg_ref, wl_ref, bl_ref, lsm_ref, sm_ref):
    z = jnp.dot(hg_ref[...], wl_ref[...],
                preferred_element_type=jnp.float32) + bl_ref[...]
    m = jnp.max(z, axis=1, keepdims=True)
    zc = z - m
    e = jnp.exp(zc)
    ssum = jnp.sum(e, axis=1, keepdims=True)
    sm_ref[...] = e / ssum
    lsm_ref[...] = zc - jnp.log(ssum)


_head = pl.pallas_call(
    _head_body,
    out_shape=[
        jax.ShapeDtypeStruct((_B, 2), jnp.float32),
        jax.ShapeDtypeStruct((_B, 2), jnp.float32),
    ],
)


def kernel(x, edge_index, relevant_batch_indices, labels,
           W1, b1, W2, b2, W3, b3, Wlin, blin):
    src = jnp.asarray(edge_index[0], jnp.int32)
    dst = jnp.asarray(edge_index[1], jnp.int32)
    padv = jnp.full((_NE - _E,), _N, jnp.int32)
    src_p = jnp.concatenate([src, padv])
    dst_p = jnp.concatenate([dst, padv])
    x_p = jnp.pad(x, ((0, _NP - _N), (0, 0)))

    srcw = _pack_edges(src_p, _C0W, _C1W)
    dstw = _pack_edges(dst_p, _C0W, _C1W)
    if (_C0N, _C1N) == (_C0W, _C1W):
        srcn, dstn = srcw, dstw
    else:
        srcn = _pack_edges(src_p, _C0N, _C1N)
        dstn = _pack_edges(dst_p, _C0N, _C1N)

    W1p = W1[:, _PERMW]
    W2p = W2[:, _PERMN]
    W3p = W3[:, _PERMN]

    degp = _deg_kernel(dst_p)
    g1, dinv = _scale_in(degp, x_p, W1p)
    p1 = _scatter_w(_to_i32(g1), srcw, dstw)
    g2 = _combine12(p1, g1, dinv, b1.reshape(1, -1), W2p)
    p2 = _scatter_n(_to_i32(g2), srcn, dstn)
    g3 = _combine23(p2, g2, dinv, b2.reshape(1, -1), W3p)
    p3 = _scatter_n(_to_i32(g3), srcn, dstn)
    h3 = _combine_out(p3, g3, dinv, b3.reshape(1, -1))
    hg = _gather_rows(h3, jnp.asarray(relevant_batch_indices, jnp.int32))
    lsm, sm = _head(hg, Wlin, blin.reshape(1, -1))
    return (lsm, sm)


# speedup vs baseline: 1.4635x; 1.4635x over previous
import functools

import jax
import jax.numpy as jnp
import numpy as np
from jax import lax
from jax.experimental import pallas as pl
from jax.experimental.pallas import tpu as pltpu
from jax.experimental.pallas import tpu_sc as plsc

_N = 10000
_NP = 10240
_E = 320000
_F = 128
_W = 128
_HQ = 32
_NC = 2
_NS = 16
_NW = _NC * _NS
_CH = 128
_CHUNKS = 80
_EPW = _CHUNKS * _CH
_NE = _NW * _EPW
_RPS = _NP // _NS
_BLK = 1024
_B = 1024
_BPW = _B // _NW
_ECH = 1280

_mesh = plsc.VectorSubcoreMesh(core_axis_name="c", subcore_axis_name="s")


@functools.partial(
    pl.kernel,
    out_type=jax.ShapeDtypeStruct((_NW, _NP), jnp.float32),
    mesh=_mesh,
    scratch_types=[
        pltpu.VMEM((_NP,), jnp.float32),
        pltpu.VMEM((_ECH,), jnp.int32),
    ],
    compiler_params=pltpu.CompilerParams(needs_layout_passes=False),
)
def _deg_kernel(dst_hbm, out_hbm, hist, dst_v):
    c = lax.axis_index("c")
    s = lax.axis_index("s")
    wid = c * _NS + s
    zv = jnp.zeros((16,), jnp.float32)

    def zloop(i, carry):
        hist[pl.ds(i * 16, 16)] = zv
        return carry

    lax.fori_loop(0, _NP // 16, zloop, 0)

    ones = jnp.ones((16,), jnp.float32)
    base = wid * _EPW

    def outer(t, carry):
        pltpu.sync_copy(dst_hbm.at[pl.ds(base + t * _ECH, _ECH)], dst_v)

        def inner(j, carry2):
            idx = dst_v[pl.ds(j * 16, 16)]
            plsc.addupdate_scatter(hist, [idx], ones)
            return carry2

        lax.fori_loop(0, _ECH // 16, inner, 0)
        return carry

    lax.fori_loop(0, _EPW // _ECH, outer, 0)
    pltpu.sync_copy(hist, out_hbm.at[wid])


def _make_scatter(width, c0_chunks, c1_chunks):
    w32 = width // 2
    cmax = max(c0_chunks, c1_chunks)

    @functools.partial(
        pl.kernel,
        out_type=jax.ShapeDtypeStruct((_NC, _NP, width), jnp.float32),
        mesh=_mesh,
        scratch_types=[
            pltpu.VMEM_SHARED((_NP, width), jnp.float32),
            pltpu.VMEM((cmax, _CH), jnp.int32),
            pltpu.VMEM((_CH,), jnp.int32),
            pltpu.VMEM((_CH,), jnp.int32),
            pltpu.VMEM((_CH, w32), jnp.uint32),
            pltpu.VMEM((_CH, w32), jnp.uint32),
            pltpu.VMEM((_CH, width), jnp.float32),
            pltpu.SemaphoreType.DMA,
            pltpu.SemaphoreType.DMA,
            pltpu.SemaphoreType.DMA,
            pltpu.SemaphoreType.DMA,
        ],
        compiler_params=pltpu.CompilerParams(
            use_tc_tiling_on_sc=False, needs_layout_passes=False),
    )
    def _scat(g_hbm, src_hbm, dst_hbm, out_hbm, acc, srcs, dstb0, dstb1,
              rows0, rows1, rowsf, gsem0, gsem1, dsem0, dsem1):
        c = lax.axis_index("c")
        s = lax.axis_index("s")
        wid = c * _NS + s
        nchunks = jnp.where(c == 0, c0_chunks, c1_chunks)

        pltpu.sync_copy(src_hbm.at[wid], srcs)

        zv = jnp.zeros((16,), jnp.float32)
        for i in range(16):
            for j in range(width // 16):
                rowsf[i, 16 * j:16 * (j + 1)] = zv

        def zloop(k, carry):
            pltpu.sync_copy(rowsf.at[pl.ds(0, 16)],
                            acc.at[pl.ds(s * _RPS + k * 16, 16)])
            return carry

        lax.fori_loop(0, _RPS // 16, zloop, 0)
        plsc.subcore_barrier()

        pltpu.async_copy(dst_hbm.at[wid, 0], dstb0, dsem0)
        pltpu.async_copy(g_hbm.at[srcs.at[0]], rows0, gsem0)

        def body(i, carry):
            for b, rows, gsem, dstb, dsem, orows, ogsem, odstb, odsem in (
                (0, rows0, gsem0, dstb0, dsem0, rows1, gsem1, dstb1, dsem1),
                (1, rows1, gsem1, dstb1, dsem1, rows0, gsem0, dstb0, dsem0),
            ):
                t = 2 * i + b
                pltpu.make_async_copy(dst_hbm.at[wid, t], dstb, dsem).wait()
                pltpu.make_async_copy(g_hbm.at[srcs.at[t]], rows, gsem).wait()
                nt = t + 1

                @pl.when(nt < nchunks)
                def _():
                    pltpu.async_copy(dst_hbm.at[wid, nt], odstb, odsem)
                    pltpu.async_copy(g_hbm.at[srcs.at[nt]], orows, ogsem)

                def conv(r, carry2):
                    for j in range(w32 // 16):
                        v = rows[r, 16 * j:16 * (j + 1)]
                        rowsf[r, 32 * j:32 * j + 16] = plsc.bitcast(
                            v << 16, jnp.float32)
                        rowsf[r, 32 * j + 16:32 * j + 32] = plsc.bitcast(
                            v & jnp.uint32(0xFFFF0000), jnp.float32)
                    return carry2

                lax.fori_loop(0, _CH, conv, 0)
                pltpu.sync_copy(rowsf, acc.at[dstb], add=True)
            return carry

        lax.fori_loop(0, nchunks // 2, body, 0)
        plsc.subcore_barrier()
        pltpu.sync_copy(acc.at[pl.ds(s * _RPS, _RPS)],
                        out_hbm.at[c, pl.ds(s * _RPS, _RPS)])

    return _scat


_C0W, _C1W = 80, 80
_C0N, _C1N = 80, 80

_scatter_w = _make_scatter(_W, _C0W, _C1W)
_scatter_n = _make_scatter(_HQ, _C0N, _C1N)


def _pack_words(hm):
    w = hm.shape[1]
    words = []
    for j in range(w // 32):
        a = hm[:, 32 * j:32 * j + 16]
        b = hm[:, 32 * j + 16:32 * j + 32]
        au = jax.lax.bitcast_convert_type(
            a.astype(jnp.bfloat16).astype(jnp.float32), jnp.uint32)
        bu = jax.lax.bitcast_convert_type(
            b.astype(jnp.bfloat16).astype(jnp.float32), jnp.uint32)
        words.append((au >> 16) | (bu & jnp.uint32(0xFFFF0000)))
    return jnp.concatenate(words, axis=1)


def _unpack_words(gw):
    w2 = gw.shape[1]
    cols = []
    for j in range(w2 // 16):
        v = gw[:, 16 * j:16 * j + 16]
        cols.append(jax.lax.bitcast_convert_type(v << 16, jnp.float32))
        cols.append(jax.lax.bitcast_convert_type(
            v & jnp.uint32(0xFFFF0000), jnp.float32))
    return jnp.concatenate(cols, axis=1)


def _pack_edges(v, c0, c1):
    cmax = max(c0, c1)
    t0 = _NS * c0 * _CH
    a = v[:t0].reshape(_NS, c0, _CH)
    b = v[t0:].reshape(_NS, c1, _CH)
    a = jnp.pad(a, ((0, 0), (0, cmax - c0), (0, 0)), constant_values=_N)
    b = jnp.pad(b, ((0, 0), (0, cmax - c1), (0, 0)), constant_values=_N)
    return jnp.concatenate([a, b], axis=0)


@functools.partial(
    pl.kernel,
    out_type=jax.ShapeDtypeStruct((_B, _HQ), jnp.float32),
    mesh=_mesh,
    scratch_types=[
        pltpu.VMEM((_BPW,), jnp.int32),
        pltpu.VMEM((_BPW, _HQ), jnp.float32),
        pltpu.SemaphoreType.DMA,
    ],
    compiler_params=pltpu.CompilerParams(use_tc_tiling_on_sc=False),
)
def _gather_rows(h_hbm, idx_hbm, out_hbm, idx_v, rows_v, sem):
    c = lax.axis_index("c")
    s = lax.axis_index("s")
    wid = c * _NS + s
    pltpu.sync_copy(idx_hbm.at[pl.ds(wid * _BPW, _BPW)], idx_v)
    pltpu.async_copy(h_hbm.at[idx_v], rows_v, sem).wait()
    pltpu.sync_copy(rows_v, out_hbm.at[pl.ds(wid * _BPW, _BPW)])


def _scale_in_body(degp_ref, x_ref, w_ref, g_ref, dinv_ref):
    deg = jnp.sum(degp_ref[...], axis=0)[:, None] + 1.0
    dinv = lax.rsqrt(deg)
    i = pl.program_id(0)
    rows = i * _BLK + lax.broadcasted_iota(jnp.int32, (_BLK, 1), 0)
    h = jnp.dot(x_ref[...], w_ref[...], preferred_element_type=jnp.float32)
    g_ref[...] = _pack_words(jnp.where(rows < _N, h * dinv, 0.0))
    dinv_ref[...] = dinv


_scale_in = pl.pallas_call(
    _scale_in_body,
    grid=(_NP // _BLK,),
    in_specs=[
        pl.BlockSpec((_NW, _BLK), lambda i: (0, i)),
        pl.BlockSpec((_BLK, _F), lambda i: (i, 0)),
        pl.BlockSpec((_F, _W), lambda i: (0, 0)),
    ],
    out_specs=[
        pl.BlockSpec((_BLK, _W // 2), lambda i: (i, 0)),
        pl.BlockSpec((_BLK, 1), lambda i: (i, 0)),
    ],
    out_shape=[
        jax.ShapeDtypeStruct((_NP, _W // 2), jnp.uint32),
        jax.ShapeDtypeStruct((_NP, 1), jnp.float32),
    ],
)


def _make_combine_matmul(din, dout):
    def body(p_ref, g_ref, dinv_ref, b_ref, w_ref, out_ref):
        i = pl.program_id(0)
        p = p_ref[...]
        dinv = dinv_ref[...]
        g = _unpack_words(g_ref[...])
        u = dinv * (p[0] + p[1] + g) + b_ref[...]
        rows = i * _BLK + lax.broadcasted_iota(jnp.int32, (_BLK, 1), 0)
        v = jnp.where(rows < _N, jnp.maximum(u, 0.0), 0.0)
        out = jnp.dot(v, w_ref[...], preferred_element_type=jnp.float32)
        out_ref[...] = _pack_words(out * dinv)

    return pl.pallas_call(
        body,
        grid=(_NP // _BLK,),
        in_specs=[
            pl.BlockSpec((2, _BLK, din), lambda i: (0, i, 0)),
            pl.BlockSpec((_BLK, din // 2), lambda i: (i, 0)),
            pl.BlockSpec((_BLK, 1), lambda i: (i, 0)),
            pl.BlockSpec((1, din), lambda i: (0, 0)),
            pl.BlockSpec((din, dout), lambda i: (0, 0)),
        ],
        out_specs=pl.BlockSpec((_BLK, dout // 2), lambda i: (i, 0)),
        out_shape=jax.ShapeDtypeStruct((_NP, dout // 2), jnp.uint32),
    )


_combine12 = _make_combine_matmul(_W, _HQ)
_combine23 = _make_combine_matmul(_HQ, _HQ)


def _combine_out_body(p_ref, g_ref, dinv_ref, b_ref, out_ref):
    p = p_ref[...]
    g = _unpack_words(g_ref[...])
    out_ref[...] = dinv_ref[...] * (p[0] + p[1] + g) + b_ref[...]


_combine_out = pl.pallas_call(
    _combine_out_body,
    grid=(_NP // _BLK,),
    in_specs=[
        pl.BlockSpec((2, _BLK, _HQ), lambda i: (0, i, 0)),
        pl.BlockSpec((_BLK, _HQ // 2), lambda i: (i, 0)),
        pl.BlockSpec((_BLK, 1), lambda i: (i, 0)),
        pl.BlockSpec((1, _HQ), lambda i: (0, 0)),
    ],
    out_specs=pl.BlockSpec((_BLK, _HQ), lambda i: (i, 0)),
    out_shape=jax.ShapeDtypeStruct((_NP, _HQ), jnp.float32),
)


def _head_body(hg_ref, wl_ref, bl_ref, lsm_ref, sm_ref):
    z = jnp.dot(hg_ref[...], wl_ref[...],
                preferred_element_type=jnp.float32) + bl_ref[...]
    m = jnp.max(z, axis=1, keepdims=True)
    zc = z - m
    e = jnp.exp(zc)
    ssum = jnp.sum(e, axis=1, keepdims=True)
    sm_ref[...] = e / ssum
    lsm_ref[...] = zc - jnp.log(ssum)


_head = pl.pallas_call(
    _head_body,
    out_shape=[
        jax.ShapeDtypeStruct((_B, 2), jnp.float32),
        jax.ShapeDtypeStruct((_B, 2), jnp.float32),
    ],
)


def kernel(x, edge_index, relevant_batch_indices, labels,
           W1, b1, W2, b2, W3, b3, Wlin, blin):
    src = jnp.asarray(edge_index[0], jnp.int32)
    dst = jnp.asarray(edge_index[1], jnp.int32)
    padv = jnp.full((_NE - _E,), _N, jnp.int32)
    src_p = jnp.concatenate([src, padv])
    dst_p = jnp.concatenate([dst, padv])
    x_p = jnp.pad(x, ((0, _NP - _N), (0, 0)))

    srcw = _pack_edges(src_p, _C0W, _C1W)
    dstw = _pack_edges(dst_p, _C0W, _C1W)
    if (_C0N, _C1N) == (_C0W, _C1W):
        srcn, dstn = srcw, dstw
    else:
        srcn = _pack_edges(src_p, _C0N, _C1N)
        dstn = _pack_edges(dst_p, _C0N, _C1N)

    degp = _deg_kernel(dst_p)
    g1, dinv = _scale_in(degp, x_p, W1)
    p1 = _scatter_w(g1, srcw, dstw)
    g2 = _combine12(p1, g1, dinv, b1.reshape(1, -1), W2)
    p2 = _scatter_n(g2, srcn, dstn)
    g3 = _combine23(p2, g2, dinv, b2.reshape(1, -1), W3)
    p3 = _scatter_n(g3, srcn, dstn)
    h3 = _combine_out(p3, g3, dinv, b3.reshape(1, -1))
    hg = _gather_rows(h3, jnp.asarray(relevant_batch_indices, jnp.int32))
    lsm, sm = _head(hg, Wlin, blin.reshape(1, -1))
    return (lsm, sm)


# speedup vs baseline: 1.4969x; 1.0228x over previous
import functools

import jax
import jax.numpy as jnp
import numpy as np
from jax import lax
from jax.experimental import pallas as pl
from jax.experimental.pallas import tpu as pltpu
from jax.experimental.pallas import tpu_sc as plsc

_N = 10000
_NP = 10240
_E = 320000
_F = 128
_W = 128
_HQ = 32
_NC = 2
_NS = 16
_NW = _NC * _NS
_CH = 128
_CHUNKS = 80
_EPW = _CHUNKS * _CH
_NE = _NW * _EPW
_RPS = _NP // _NS
_NPA = 10016
_RPA = _NPA // _NS
_BLK = 1024
_B = 1024
_BPW = _B // _NW
_ECH = 1280

_mesh = plsc.VectorSubcoreMesh(core_axis_name="c", subcore_axis_name="s")


@functools.partial(
    pl.kernel,
    out_type=jax.ShapeDtypeStruct((_NW, _NP), jnp.float32),
    mesh=_mesh,
    scratch_types=[
        pltpu.VMEM((_NP,), jnp.float32),
        pltpu.VMEM((_ECH,), jnp.int32),
    ],
    compiler_params=pltpu.CompilerParams(needs_layout_passes=False),
)
def _deg_kernel(dst_hbm, out_hbm, hist, dst_v):
    c = lax.axis_index("c")
    s = lax.axis_index("s")
    wid = c * _NS + s
    zv = jnp.zeros((16,), jnp.float32)

    def zloop(i, carry):
        hist[pl.ds(i * 16, 16)] = zv
        return carry

    lax.fori_loop(0, _NP // 16, zloop, 0)

    ones = jnp.ones((16,), jnp.float32)
    base = wid * _EPW

    def outer(t, carry):
        pltpu.sync_copy(dst_hbm.at[pl.ds(base + t * _ECH, _ECH)], dst_v)

        def inner(j, carry2):
            idx = dst_v[pl.ds(j * 16, 16)]
            plsc.addupdate_scatter(hist, [idx], ones)
            return carry2

        lax.fori_loop(0, _ECH // 16, inner, 0)
        return carry

    lax.fori_loop(0, _EPW // _ECH, outer, 0)
    pltpu.sync_copy(hist, out_hbm.at[wid])


def _make_scatter(width, c0_chunks, c1_chunks):
    w32 = width // 2

    @functools.partial(
        pl.kernel,
        out_type=jax.ShapeDtypeStruct((_NC, _NP, width), jnp.float32),
        mesh=_mesh,
        scratch_types=[
            pltpu.VMEM_SHARED((_NPA, width), jnp.float32),
            [pltpu.VMEM((_CH,), jnp.int32) for _ in range(2)],
            [pltpu.VMEM((_CH,), jnp.int32) for _ in range(4)],
            [pltpu.VMEM((_CH, w32), jnp.uint32) for _ in range(2)],
            [pltpu.VMEM((_CH, width), jnp.float32) for _ in range(2)],
            [pltpu.SemaphoreType.DMA for _ in range(2)],
            [pltpu.SemaphoreType.DMA for _ in range(4)],
            [pltpu.SemaphoreType.DMA for _ in range(2)],
            [pltpu.SemaphoreType.DMA for _ in range(2)],
        ],
        compiler_params=pltpu.CompilerParams(
            use_tc_tiling_on_sc=False, needs_layout_passes=False),
    )
    def _scat(g_hbm, src_hbm, dst_hbm, out_hbm, acc, srcb, dstb, rows, rowsf,
              isem, dsem, gsem, ssem):
        c = lax.axis_index("c")
        s = lax.axis_index("s")
        wid = c * _NS + s
        nchunks = jnp.where(c == 0, c0_chunks, c1_chunks)

        zv = jnp.zeros((16,), jnp.float32)
        for i in range(16):
            for j in range(width // 16):
                rowsf[0][i, 16 * j:16 * (j + 1)] = zv

        def zloop(k, carry):
            pltpu.sync_copy(rowsf[0].at[pl.ds(0, 16)],
                            acc.at[pl.ds(s * _RPA + k * 16, 16)])
            return carry

        lax.fori_loop(0, _RPA // 16, zloop, 0)
        pltpu.sync_copy(rowsf[0].at[pl.ds(0, _RPA - 16 * (_RPA // 16))],
                        acc.at[pl.ds(s * _RPA + 16 * (_RPA // 16),
                                     _RPA - 16 * (_RPA // 16))])

        for a in range(2):
            pltpu.async_copy(src_hbm.at[wid, a], srcb[a], isem[a])
            pltpu.async_copy(dst_hbm.at[wid, a], dstb[a], dsem[a])
        pltpu.make_async_copy(src_hbm.at[wid, 0], srcb[0], isem[0]).wait()
        pltpu.async_copy(g_hbm.at[srcb[0]], rows[0], gsem[0])
        plsc.subcore_barrier()

        def body(i, carry):
            for k in range(4):
                b = k % 2
                a = k
                t = 4 * i + k
                pltpu.make_async_copy(
                    dst_hbm.at[wid, t], dstb[a], dsem[a]).wait()
                pltpu.make_async_copy(
                    g_hbm.at[srcb[b]], rows[b], gsem[b]).wait()

                @pl.when(t + 2 < nchunks)
                def _():
                    pltpu.async_copy(src_hbm.at[wid, t + 2], srcb[b], isem[b])

                @pl.when(t + 1 < nchunks)
                def _():
                    pltpu.make_async_copy(
                        src_hbm.at[wid, t + 1], srcb[1 - b],
                        isem[1 - b]).wait()
                    pltpu.async_copy(
                        g_hbm.at[srcb[1 - b]], rows[1 - b], gsem[1 - b])

                @pl.when(t >= 2)
                def _():
                    pltpu.make_async_copy(
                        out_hbm.at[c, pl.ds(0, _CH)], rowsf[b],
                        ssem[b]).wait()

                @pl.when(t + 2 < nchunks)
                def _():
                    pltpu.async_copy(
                        dst_hbm.at[wid, t + 2], dstb[(a + 2) % 4],
                        dsem[(a + 2) % 4])

                def conv(r, carry2):
                    for j in range(w32 // 16):
                        v = rows[b][r, 16 * j:16 * (j + 1)]
                        rowsf[b][r, 32 * j:32 * j + 16] = plsc.bitcast(
                            v << 16, jnp.float32)
                        rowsf[b][r, 32 * j + 16:32 * j + 32] = plsc.bitcast(
                            v & jnp.uint32(0xFFFF0000), jnp.float32)
                    return carry2

                lax.fori_loop(0, _CH, conv, 0)
                pltpu.async_copy(rowsf[b], acc.at[dstb[a]], ssem[b], add=True)
            return carry

        lax.fori_loop(0, nchunks // 4, body, 0)
        for b in range(2):
            pltpu.make_async_copy(
                out_hbm.at[c, pl.ds(0, _CH)], rowsf[b], ssem[b]).wait()
        plsc.subcore_barrier()
        pltpu.sync_copy(acc.at[pl.ds(s * _RPA, _RPA)],
                        out_hbm.at[c, pl.ds(s * _RPA, _RPA)])

    return _scat


_C0W, _C1W = 80, 80
_C0N, _C1N = 80, 80

_scatter_w = _make_scatter(_W, _C0W, _C1W)
_scatter_n = _make_scatter(_HQ, _C0N, _C1N)


def _pack_words(hm):
    w = hm.shape[1]
    words = []
    for j in range(w // 32):
        a = hm[:, 32 * j:32 * j + 16]
        b = hm[:, 32 * j + 16:32 * j + 32]
        au = jax.lax.bitcast_convert_type(
            a.astype(jnp.bfloat16).astype(jnp.float32), jnp.uint32)
        bu = jax.lax.bitcast_convert_type(
            b.astype(jnp.bfloat16).astype(jnp.float32), jnp.uint32)
        words.append((au >> 16) | (bu & jnp.uint32(0xFFFF0000)))
    return jnp.concatenate(words, axis=1)


def _unpack_words(gw):
    w2 = gw.shape[1]
    cols = []
    for j in range(w2 // 16):
        v = gw[:, 16 * j:16 * j + 16]
        cols.append(jax.lax.bitcast_convert_type(v << 16, jnp.float32))
        cols.append(jax.lax.bitcast_convert_type(
            v & jnp.uint32(0xFFFF0000), jnp.float32))
    return jnp.concatenate(cols, axis=1)


def _pack_edges(v, c0, c1):
    cmax = max(c0, c1)
    t0 = _NS * c0 * _CH
    a = v[:t0].reshape(_NS, c0, _CH)
    b = v[t0:].reshape(_NS, c1, _CH)
    a = jnp.pad(a, ((0, 0), (0, cmax - c0), (0, 0)), constant_values=_N)
    b = jnp.pad(b, ((0, 0), (0, cmax - c1), (0, 0)), constant_values=_N)
    return jnp.concatenate([a, b], axis=0)


@functools.partial(
    pl.kernel,
    out_type=jax.ShapeDtypeStruct((_B, _HQ), jnp.float32),
    mesh=_mesh,
    scratch_types=[
        pltpu.VMEM((_BPW,), jnp.int32),
        pltpu.VMEM((_BPW, _HQ), jnp.float32),
        pltpu.SemaphoreType.DMA,
    ],
    compiler_params=pltpu.CompilerParams(use_tc_tiling_on_sc=False),
)
def _gather_rows(h_hbm, idx_hbm, out_hbm, idx_v, rows_v, sem):
    c = lax.axis_index("c")
    s = lax.axis_index("s")
    wid = c * _NS + s
    pltpu.sync_copy(idx_hbm.at[pl.ds(wid * _BPW, _BPW)], idx_v)
    pltpu.async_copy(h_hbm.at[idx_v], rows_v, sem).wait()
    pltpu.sync_copy(rows_v, out_hbm.at[pl.ds(wid * _BPW, _BPW)])


def _scale_in_body(degp_ref, x_ref, w_ref, g_ref, dinv_ref):
    deg = jnp.sum(degp_ref[...], axis=0)[:, None] + 1.0
    dinv = lax.rsqrt(deg)
    i = pl.program_id(0)
    rows = i * _BLK + lax.broadcasted_iota(jnp.int32, (_BLK, 1), 0)
    h = jnp.dot(x_ref[...], w_ref[...], preferred_element_type=jnp.float32)
    g_ref[...] = _pack_words(jnp.where(rows < _N, h * dinv, 0.0))
    dinv_ref[...] = dinv


_scale_in = pl.pallas_call(
    _scale_in_body,
    grid=(_NP // _BLK,),
    in_specs=[
        pl.BlockSpec((_NW, _BLK), lambda i: (0, i)),
        pl.BlockSpec((_BLK, _F), lambda i: (i, 0)),
        pl.BlockSpec((_F, _W), lambda i: (0, 0)),
    ],
    out_specs=[
        pl.BlockSpec((_BLK, _W // 2), lambda i: (i, 0)),
        pl.BlockSpec((_BLK, 1), lambda i: (i, 0)),
    ],
    out_shape=[
        jax.ShapeDtypeStruct((_NP, _W // 2), jnp.uint32),
        jax.ShapeDtypeStruct((_NP, 1), jnp.float32),
    ],
)


def _make_combine_matmul(din, dout):
    def body(p_ref, g_ref, dinv_ref, b_ref, w_ref, out_ref):
        i = pl.program_id(0)
        p = p_ref[...]
        dinv = dinv_ref[...]
        g = _unpack_words(g_ref[...])
        u = dinv * (p[0] + p[1] + g) + b_ref[...]
        rows = i * _BLK + lax.broadcasted_iota(jnp.int32, (_BLK, 1), 0)
        v = jnp.where(rows < _N, jnp.maximum(u, 0.0), 0.0)
        out = jnp.dot(v, w_ref[...], preferred_element_type=jnp.float32)
        out_ref[...] = _pack_words(out * dinv)

    return pl.pallas_call(
        body,
        grid=(_NP // _BLK,),
        in_specs=[
            pl.BlockSpec((2, _BLK, din), lambda i: (0, i, 0)),
            pl.BlockSpec((_BLK, din // 2), lambda i: (i, 0)),
            pl.BlockSpec((_BLK, 1), lambda i: (i, 0)),
            pl.BlockSpec((1, din), lambda i: (0, 0)),
            pl.BlockSpec((din, dout), lambda i: (0, 0)),
        ],
        out_specs=pl.BlockSpec((_BLK, dout // 2), lambda i: (i, 0)),
        out_shape=jax.ShapeDtypeStruct((_NP, dout // 2), jnp.uint32),
    )


_combine12 = _make_combine_matmul(_W, _HQ)
_combine23 = _make_combine_matmul(_HQ, _HQ)


def _combine_out_body(p_ref, g_ref, dinv_ref, b_ref, out_ref):
    p = p_ref[...]
    g = _unpack_words(g_ref[...])
    out_ref[...] = dinv_ref[...] * (p[0] + p[1] + g) + b_ref[...]


_combine_out = pl.pallas_call(
    _combine_out_body,
    grid=(_NP // _BLK,),
    in_specs=[
        pl.BlockSpec((2, _BLK, _HQ), lambda i: (0, i, 0)),
        pl.BlockSpec((_BLK, _HQ // 2), lambda i: (i, 0)),
        pl.BlockSpec((_BLK, 1), lambda i: (i, 0)),
        pl.BlockSpec((1, _HQ), lambda i: (0, 0)),
    ],
    out_specs=pl.BlockSpec((_BLK, _HQ), lambda i: (i, 0)),
    out_shape=jax.ShapeDtypeStruct((_NP, _HQ), jnp.float32),
)


def _head_body(hg_ref, wl_ref, bl_ref, lsm_ref, sm_ref):
    z = jnp.dot(hg_ref[...], wl_ref[...],
                preferred_element_type=jnp.float32) + bl_ref[...]
    m = jnp.max(z, axis=1, keepdims=True)
    zc = z - m
    e = jnp.exp(zc)
    ssum = jnp.sum(e, axis=1, keepdims=True)
    sm_ref[...] = e / ssum
    lsm_ref[...] = zc - jnp.log(ssum)


_head = pl.pallas_call(
    _head_body,
    out_shape=[
        jax.ShapeDtypeStruct((_B, 2), jnp.float32),
        jax.ShapeDtypeStruct((_B, 2), jnp.float32),
    ],
)


def kernel(x, edge_index, relevant_batch_indices, labels,
           W1, b1, W2, b2, W3, b3, Wlin, blin):
    src = jnp.asarray(edge_index[0], jnp.int32)
    dst = jnp.asarray(edge_index[1], jnp.int32)
    padv = jnp.full((_NE - _E,), _N, jnp.int32)
    src_p = jnp.concatenate([src, padv])
    dst_p = jnp.concatenate([dst, padv])
    x_p = jnp.pad(x, ((0, _NP - _N), (0, 0)))

    srcw = _pack_edges(src_p, _C0W, _C1W)
    dstw = _pack_edges(dst_p, _C0W, _C1W)
    if (_C0N, _C1N) == (_C0W, _C1W):
        srcn, dstn = srcw, dstw
    else:
        srcn = _pack_edges(src_p, _C0N, _C1N)
        dstn = _pack_edges(dst_p, _C0N, _C1N)

    degp = _deg_kernel(dst_p)
    g1, dinv = _scale_in(degp, x_p, W1)
    p1 = _scatter_w(g1, srcw, dstw)
    g2 = _combine12(p1, g1, dinv, b1.reshape(1, -1), W2)
    p2 = _scatter_n(g2, srcn, dstn)
    g3 = _combine23(p2, g2, dinv, b2.reshape(1, -1), W3)
    p3 = _scatter_n(g3, srcn, dstn)
    h3 = _combine_out(p3, g3, dinv, b3.reshape(1, -1))
    hg = _gather_rows(h3, jnp.asarray(relevant_batch_indices, jnp.int32))
    lsm, sm = _head(hg, Wlin, blin.reshape(1, -1))
    return (lsm, sm)
